# Initial kernel scaffold; baseline (speedup 1.0000x reference)
import jax
import jax.numpy as jnp
from jax.experimental import pallas as pl


def kernel(x, neurons, Wq, bq, Wk, bk, Wv, bv, Wp, bp):
    raise NotImplementedError("write your pallas kernel here")


# speedup vs baseline: 2.0634x; 2.0634x over previous
import functools
import math

import jax
import jax.numpy as jnp
from jax.experimental import pallas as pl
from jax.experimental.pallas import tpu as pltpu

H = 16
K = 8


def _qkv_body(x_ref, w_ref, b_ref, o_ref):
    o_ref[...] = (
        jnp.dot(x_ref[...], w_ref[...], preferred_element_type=jnp.float32)
        + b_ref[...]
    )


def _attn_body(q_ref, k_ref, v_ref, o_ref, *, scale):
    s = jax.lax.dot_general(
        q_ref[0], k_ref[0], (((1,), (1,)), ((), ())),
        preferred_element_type=jnp.float32,
    ) * scale
    skv = s.shape[1]
    half = skv // 2
    v = v_ref[0]
    s1 = s[:, :half]
    s2 = s[:, half:]
    v1 = v[:half]
    v2 = v[half:]

    m1 = jnp.max(s1, axis=1, keepdims=True)
    e1 = jnp.exp(s1 - m1)
    bs1 = jnp.sum(e1, axis=1, keepdims=True)
    o1 = jnp.dot(e1, v1, preferred_element_type=jnp.float32)
    out1 = o1 * (1.0 / bs1)

    m2 = jnp.max(s2, axis=1, keepdims=True)
    mnew = jnp.maximum(m1, m2)
    delta = jnp.where(m1 == mnew, 0.0, m1 - mnew)
    ed = jnp.exp(delta)
    e2 = jnp.exp(s2 - mnew)
    bs2 = jnp.sum(e2, axis=1, keepdims=True)
    resc = ed * bs1
    sum2 = resc + bs2
    acc = resc * out1
    o2 = jnp.dot(e2, v2, preferred_element_type=jnp.float32) + acc
    o_ref[0] = o2 * (1.0 / sum2)


def _router_body(x_ref, c_ref, wp_ref, bp_ref, n_ref,
                 idx_ref, tw_ref, sel_ref, out_ref, *, n_neurons):
    xb = x_ref[...]
    cb = c_ref[...]
    comb = jnp.concatenate([xb, cb], axis=1)
    logits = (
        jnp.dot(comb, wp_ref[...], preferred_element_type=jnp.float32)
        + bp_ref[...]
    )
    m = jnp.max(logits, axis=1, keepdims=True)
    e = jnp.exp(logits - m)
    w = e / jnp.sum(e, axis=1, keepdims=True)
    token_s = jax.lax.dot_general(
        xb, n_ref[...], (((1,), (1,)), ((), ())),
        preferred_element_type=jnp.float32,
    )
    ctx_s = jax.lax.dot_general(
        cb, n_ref[...], (((1,), (1,)), ((), ())),
        preferred_element_type=jnp.float32,
    )
    scores = w[:, 0:1] * token_s + w[:, 1:2] * ctx_s

    bt = scores.shape[0]
    iota_n = jax.lax.broadcasted_iota(jnp.int32, (bt, n_neurons), 1)
    iota_k = jax.lax.broadcasted_iota(jnp.int32, (bt, K), 1)
    s = scores
    tv = jnp.zeros((bt, K), dtype=jnp.float32)
    ti = jnp.zeros((bt, K), dtype=jnp.int32)
    picks = []
    for k in range(K):
        mk = jnp.max(s, axis=1, keepdims=True)
        ak = jnp.min(
            jnp.where(s == mk, iota_n, n_neurons), axis=1, keepdims=True
        )
        picks.append((ak, mk))
        tv = jnp.where(iota_k == k, mk, tv)
        ti = jnp.where(iota_k == k, ak, ti)
        s = jnp.where(iota_n == ak, -jnp.inf, s)

    ew = jnp.exp(tv - tv[:, 0:1])
    tw = ew / jnp.sum(ew, axis=1, keepdims=True)

    idx_ref[...] = ti
    tw_ref[...] = tw

    sel = jnp.zeros((bt, n_neurons), dtype=jnp.float32)
    for k in range(K):
        ak, _ = picks[k]
        sel = sel + jnp.where(iota_n == ak, tw[:, k:k + 1], 0.0)
    sel_ref[...] = sel
    out_ref[...] = jnp.dot(sel, n_ref[...], preferred_element_type=jnp.float32, precision=jax.lax.Precision.HIGHEST)


def kernel(x, neurons, Wq, bq, Wk, bk, Wv, bv, Wp, bp):
    Bsz, S, D = x.shape
    dh = D // H
    n_neurons = neurons.shape[0]
    x2 = x.reshape(S, D)

    Wqkv = jnp.concatenate([Wq, Wk, Wv], axis=1)
    bqkv = jnp.concatenate([bq, bk, bv]).reshape(1, 3 * D)

    BT = min(256, S)
    nblk = S // BT

    qkv = pl.pallas_call(
        _qkv_body,
        grid=(nblk,),
        in_specs=[
            pl.BlockSpec((BT, D), lambda j: (j, 0)),
            pl.BlockSpec((D, 3 * D), lambda j: (0, 0)),
            pl.BlockSpec((1, 3 * D), lambda j: (0, 0)),
        ],
        out_specs=pl.BlockSpec((BT, 3 * D), lambda j: (j, 0)),
        out_shape=jax.ShapeDtypeStruct((S, 3 * D), jnp.float32),
    )(x2, Wqkv, bqkv)

    q3 = qkv[:, :D].reshape(S, H, dh).transpose(1, 0, 2)
    k3 = qkv[:, D:2 * D].reshape(S, H, dh).transpose(1, 0, 2)
    v3 = qkv[:, 2 * D:].reshape(S, H, dh).transpose(1, 0, 2)

    ctx3 = pl.pallas_call(
        functools.partial(_attn_body, scale=1.0 / math.sqrt(dh)),
        grid=(H, nblk),
        in_specs=[
            pl.BlockSpec((1, BT, dh), lambda h, j: (h, j, 0)),
            pl.BlockSpec((1, S, dh), lambda h, j: (h, 0, 0)),
            pl.BlockSpec((1, S, dh), lambda h, j: (h, 0, 0)),
        ],
        out_specs=pl.BlockSpec((1, BT, dh), lambda h, j: (h, j, 0)),
        out_shape=jax.ShapeDtypeStruct((H, S, dh), jnp.float32),
    )(q3, k3, v3)
    context = ctx3.transpose(1, 0, 2).reshape(S, D)

    bp2 = bp.reshape(1, 2)

    topk_idx, topk_w, sel, out = pl.pallas_call(
        functools.partial(_router_body, n_neurons=n_neurons),
        grid=(nblk,),
        in_specs=[
            pl.BlockSpec((BT, D), lambda j: (j, 0)),
            pl.BlockSpec((BT, D), lambda j: (j, 0)),
            pl.BlockSpec((2 * D, 2), lambda j: (0, 0)),
            pl.BlockSpec((1, 2), lambda j: (0, 0)),
            pl.BlockSpec((n_neurons, D), lambda j: (0, 0)),
        ],
        out_specs=[
            pl.BlockSpec((BT, K), lambda j: (j, 0)),
            pl.BlockSpec((BT, K), lambda j: (j, 0)),
            pl.BlockSpec((BT, n_neurons), lambda j: (j, 0)),
            pl.BlockSpec((BT, D), lambda j: (j, 0)),
        ],
        out_shape=[
            jax.ShapeDtypeStruct((S, K), jnp.int32),
            jax.ShapeDtypeStruct((S, K), jnp.float32),
            jax.ShapeDtypeStruct((S, n_neurons), jnp.float32),
            jax.ShapeDtypeStruct((S, D), jnp.float32),
        ],
    )(x2, context, Wp, bp2, neurons)

    return (
        out.reshape(Bsz, S, D),
        topk_idx.reshape(Bsz, S, K),
        topk_w.reshape(Bsz, S, K),
        sel.reshape(Bsz, S, n_neurons),
    )


# speedup vs baseline: 3.0759x; 1.4907x over previous
import functools
import math

import jax
import jax.numpy as jnp
from jax.experimental import pallas as pl
from jax.experimental.pallas import tpu as pltpu

H = 16
K = 8


def _qkv_body(x_ref, wq_ref, wk_ref, wv_ref, bq_ref, bk_ref, bv_ref,
              q_ref, k_ref, v_ref):
    xb = x_ref[...]
    q_ref[...] = jnp.dot(xb, wq_ref[...], preferred_element_type=jnp.float32) + bq_ref[...]
    k_ref[...] = jnp.dot(xb, wk_ref[...], preferred_element_type=jnp.float32) + bk_ref[...]
    v_ref[...] = jnp.dot(xb, wv_ref[...], preferred_element_type=jnp.float32) + bv_ref[...]


def _head_attn(q, k, v, scale):
    s = jax.lax.dot_general(
        q, k, (((1,), (1,)), ((), ())), preferred_element_type=jnp.float32
    ) * scale
    half = s.shape[1] // 2
    s1 = s[:, :half]
    s2 = s[:, half:]
    v1 = v[:half]
    v2 = v[half:]

    m1 = jnp.max(s1, axis=1, keepdims=True)
    e1 = jnp.exp(s1 - m1)
    bs1 = jnp.sum(e1, axis=1, keepdims=True)
    o1 = jnp.dot(e1, v1, preferred_element_type=jnp.float32)
    out1 = o1 * (1.0 / bs1)

    m2 = jnp.max(s2, axis=1, keepdims=True)
    mnew = jnp.maximum(m1, m2)
    delta = jnp.where(m1 == mnew, 0.0, m1 - mnew)
    ed = jnp.exp(delta)
    e2 = jnp.exp(s2 - mnew)
    bs2 = jnp.sum(e2, axis=1, keepdims=True)
    resc = ed * bs1
    sum2 = resc + bs2
    acc = resc * out1
    o2 = jnp.dot(e2, v2, preferred_element_type=jnp.float32) + acc
    return o2 * (1.0 / sum2)


def _attn_body(q_ref, k_ref, v_ref, o_ref, k0_s, k1_s, v0_s, v1_s, *, scale, dh):
    j = pl.program_id(1)

    @pl.when(j == 0)
    def _stash():
        kp = k_ref[...]
        vp = v_ref[...]
        k0_s[...] = kp[:, :dh]
        k1_s[...] = kp[:, dh:]
        v0_s[...] = vp[:, :dh]
        v1_s[...] = vp[:, dh:]

    qp = q_ref[...]
    c0 = _head_attn(qp[:, :dh], k0_s[...], v0_s[...], scale)
    c1 = _head_attn(qp[:, dh:], k1_s[...], v1_s[...], scale)
    o_ref[...] = jnp.concatenate([c0, c1], axis=1)


def _router_body(x_ref, c_ref, wp_ref, bp_ref, n_ref,
                 idx_ref, tw_ref, sel_ref, out_ref, *, n_neurons):
    xb = x_ref[...]
    cb = c_ref[...]
    comb = jnp.concatenate([xb, cb], axis=1)
    logits = (
        jnp.dot(comb, wp_ref[...], preferred_element_type=jnp.float32)
        + bp_ref[...]
    )
    m = jnp.max(logits, axis=1, keepdims=True)
    e = jnp.exp(logits - m)
    w = e / jnp.sum(e, axis=1, keepdims=True)
    token_s = jax.lax.dot_general(
        xb, n_ref[...], (((1,), (1,)), ((), ())),
        preferred_element_type=jnp.float32,
    )
    ctx_s = jax.lax.dot_general(
        cb, n_ref[...], (((1,), (1,)), ((), ())),
        preferred_element_type=jnp.float32,
    )
    scores = w[:, 0:1] * token_s + w[:, 1:2] * ctx_s

    bt = scores.shape[0]
    iota_n = jax.lax.broadcasted_iota(jnp.int32, (bt, n_neurons), 1)
    iota_k = jax.lax.broadcasted_iota(jnp.int32, (bt, K), 1)
    s = scores
    tv = jnp.zeros((bt, K), dtype=jnp.float32)
    ti = jnp.zeros((bt, K), dtype=jnp.int32)
    picks = []
    for k in range(K):
        mk = jnp.max(s, axis=1, keepdims=True)
        ak = jnp.min(
            jnp.where(s == mk, iota_n, n_neurons), axis=1, keepdims=True
        )
        picks.append(ak)
        tv = jnp.where(iota_k == k, mk, tv)
        ti = jnp.where(iota_k == k, ak, ti)
        s = jnp.where(iota_n == ak, -jnp.inf, s)

    ew = jnp.exp(tv - tv[:, 0:1])
    tw = ew / jnp.sum(ew, axis=1, keepdims=True)

    idx_ref[...] = ti
    tw_ref[...] = tw

    sel = jnp.zeros((bt, n_neurons), dtype=jnp.float32)
    for k in range(K):
        sel = sel + jnp.where(iota_n == picks[k], tw[:, k:k + 1], 0.0)
    sel_ref[...] = sel
    out_ref[...] = jnp.dot(sel, n_ref[...], preferred_element_type=jnp.float32,
                           precision=jax.lax.Precision.HIGHEST)


def kernel(x, neurons, Wq, bq, Wk, bk, Wv, bv, Wp, bp):
    Bsz, S, D = x.shape
    dh = D // H
    n_neurons = neurons.shape[0]
    x2 = x.reshape(S, D)

    BT = min(256, S)
    nblk = S // BT

    q2, k2, v2 = pl.pallas_call(
        _qkv_body,
        grid=(nblk,),
        in_specs=[
            pl.BlockSpec((BT, D), lambda j: (j, 0)),
            pl.BlockSpec((D, D), lambda j: (0, 0)),
            pl.BlockSpec((D, D), lambda j: (0, 0)),
            pl.BlockSpec((D, D), lambda j: (0, 0)),
            pl.BlockSpec((1, D), lambda j: (0, 0)),
            pl.BlockSpec((1, D), lambda j: (0, 0)),
            pl.BlockSpec((1, D), lambda j: (0, 0)),
        ],
        out_specs=[
            pl.BlockSpec((BT, D), lambda j: (j, 0)),
            pl.BlockSpec((BT, D), lambda j: (j, 0)),
            pl.BlockSpec((BT, D), lambda j: (j, 0)),
        ],
        out_shape=[
            jax.ShapeDtypeStruct((S, D), jnp.float32),
            jax.ShapeDtypeStruct((S, D), jnp.float32),
            jax.ShapeDtypeStruct((S, D), jnp.float32),
        ],
    )(x2, Wq, Wk, Wv, bq.reshape(1, D), bk.reshape(1, D), bv.reshape(1, D))

    hp = H // 2
    context = pl.pallas_call(
        functools.partial(_attn_body, scale=1.0 / math.sqrt(dh), dh=dh),
        grid=(hp, nblk),
        in_specs=[
            pl.BlockSpec((BT, 2 * dh), lambda h, j: (j, h)),
            pl.BlockSpec((S, 2 * dh), lambda h, j: (0, h)),
            pl.BlockSpec((S, 2 * dh), lambda h, j: (0, h)),
        ],
        out_specs=pl.BlockSpec((BT, 2 * dh), lambda h, j: (j, h)),
        out_shape=jax.ShapeDtypeStruct((S, D), jnp.float32),
        scratch_shapes=[
            pltpu.VMEM((S, dh), jnp.float32),
            pltpu.VMEM((S, dh), jnp.float32),
            pltpu.VMEM((S, dh), jnp.float32),
            pltpu.VMEM((S, dh), jnp.float32),
        ],
    )(q2, k2, v2)

    topk_idx, topk_w, sel, out = pl.pallas_call(
        functools.partial(_router_body, n_neurons=n_neurons),
        grid=(nblk,),
        in_specs=[
            pl.BlockSpec((BT, D), lambda j: (j, 0)),
            pl.BlockSpec((BT, D), lambda j: (j, 0)),
            pl.BlockSpec((2 * D, 2), lambda j: (0, 0)),
            pl.BlockSpec((1, 2), lambda j: (0, 0)),
            pl.BlockSpec((n_neurons, D), lambda j: (0, 0)),
        ],
        out_specs=[
            pl.BlockSpec((BT, K), lambda j: (j, 0)),
            pl.BlockSpec((BT, K), lambda j: (j, 0)),
            pl.BlockSpec((BT, n_neurons), lambda j: (j, 0)),
            pl.BlockSpec((BT, D), lambda j: (j, 0)),
        ],
        out_shape=[
            jax.ShapeDtypeStruct((S, K), jnp.int32),
            jax.ShapeDtypeStruct((S, K), jnp.float32),
            jax.ShapeDtypeStruct((S, n_neurons), jnp.float32),
            jax.ShapeDtypeStruct((S, D), jnp.float32),
        ],
    )(x2, context, Wp, bp.reshape(1, 2), neurons)

    return (
        out.reshape(Bsz, S, D),
        topk_idx.reshape(Bsz, S, K),
        topk_w.reshape(Bsz, S, K),
        sel.reshape(Bsz, S, n_neurons),
    )
